# Initial kernel scaffold; baseline (speedup 1.0000x reference)
import jax
import jax.numpy as jnp
from jax.experimental import pallas as pl


def kernel(x, edge_attr, W_in, b_in, Wpre, bpre, Wpost, bpost, Wp1, bp1, Wp2, bp2, edge_index):
    raise NotImplementedError("write your pallas kernel here")


# speedup vs baseline: 1.9051x; 1.9051x over previous
import functools

import jax
import jax.numpy as jnp
from jax import lax
from jax.experimental import pallas as pl
from jax.experimental.pallas import tpu as pltpu
from jax.experimental.pallas import tpu_sc as plsc

N = 10000
E = 320000
H = 128
DE = 16
L = 3
TGT = 128

NPAD = 12288
NBINS = 96
BS = 128
NBW = 3
NW = 32
KCH = 32
REG_PAD = 128
EP = E + NBINS * REG_PAD
STAGE = 1152
FLUSH = 1024
NEG = -3.0e38
POS = 3.0e38

_mesh = plsc.VectorSubcoreMesh(core_axis_name="c", subcore_axis_name="s")
_sc_params = pltpu.CompilerParams(needs_layout_passes=False)


def _wid():
    return lax.axis_index("s") * 2 + lax.axis_index("c")


def _bin_of(v16):
    return lax.shift_right_logical(v16, 7)


def _lane(ref, i):
    return ref[pl.ds(i, 16)][0]


@functools.partial(
    pl.kernel,
    mesh=_mesh,
    compiler_params=_sc_params,
    out_type=jax.ShapeDtypeStruct((NBINS, 16), jnp.int32),
    scratch_types=[
        pltpu.VMEM((2016,), jnp.int32),
        pltpu.VMEM((16,), jnp.int32),
    ],
)
def _sc_count(dst_hbm, counts_hbm, dbuf, row_v):
    w = _wid()
    mybins = [NBW * w + k for k in range(NBW)]
    CH = 2000
    nch = E // CH

    def chunk(ci, carry):
        base = pl.multiple_of(ci * CH, 8)
        pltpu.sync_copy(dst_hbm.at[pl.ds(base, CH)], dbuf.at[pl.ds(0, CH)])

        def grp(g, c2):
            v = dbuf[pl.ds(g * 16, 16)]
            b = _bin_of(v)
            return tuple(c2[k] + jnp.sum((b == mybins[k]).astype(jnp.int32))
                         for k in range(NBW))

        return lax.fori_loop(0, CH // 16, grp, carry)

    cnts = lax.fori_loop(0, nch, chunk, (0,) * NBW)
    lanes = lax.iota(jnp.int32, 16)
    for k in range(NBW):
        row_v[pl.ds(0, 16)] = jnp.where(lanes == 0, cnts[k], 0)
        pltpu.sync_copy(row_v, counts_hbm.at[mybins[k]])


@functools.partial(
    pl.kernel,
    mesh=_mesh,
    compiler_params=_sc_params,
    out_type=(
        jax.ShapeDtypeStruct((EP,), jnp.int32),
        jax.ShapeDtypeStruct((EP,), jnp.int32),
        jax.ShapeDtypeStruct((EP,), jnp.int32),
        jax.ShapeDtypeStruct((NBINS, 16), jnp.int32),
    ),
    scratch_types=[
        pltpu.VMEM((NBINS, 16), jnp.int32),
        pltpu.VMEM((2016,), jnp.int32),
        pltpu.VMEM((2016,), jnp.int32),
        pltpu.VMEM((STAGE,), jnp.int32),
        pltpu.VMEM((STAGE,), jnp.int32),
        pltpu.VMEM((STAGE,), jnp.int32),
        pltpu.VMEM((STAGE,), jnp.int32),
        pltpu.VMEM((STAGE,), jnp.int32),
        pltpu.VMEM((STAGE,), jnp.int32),
        pltpu.VMEM((STAGE,), jnp.int32),
        pltpu.VMEM((STAGE,), jnp.int32),
        pltpu.VMEM((STAGE,), jnp.int32),
        pltpu.VMEM((16,), jnp.int32),
    ],
)
def _sc_place(src_hbm, dst_hbm, counts_hbm, psrc_hbm, pdst_hbm, peid_hbm,
              meta_hbm, cnts_v, sbuf, dbuf, s0, d0, e0, s1, d1, e1,
              s2, d2, e2, row_v):
    w = _wid()
    mybins = [NBW * w + k for k in range(NBW)]
    CH = 2000
    nch = E // CH
    lanes = lax.iota(jnp.int32, 16)

    pltpu.sync_copy(counts_hbm, cnts_v)

    def scan_bins(b, carry):
        s = carry[0]
        sts = list(carry[1:1 + NBW])
        cns = list(carry[1 + NBW:])
        c = cnts_v[b, pl.ds(0, 16)][0]
        cp = (c + (REG_PAD - 1)) & (-REG_PAD)
        for k in range(NBW):
            sts[k] = jnp.where(b == mybins[k], s, sts[k])
            cns[k] = jnp.where(b == mybins[k], c, cns[k])
        return (s + cp,) + tuple(sts) + tuple(cns)

    res = lax.fori_loop(0, NBINS, scan_bins, (0,) * (1 + 2 * NBW))
    starts = res[1:1 + NBW]
    rawcnt = res[1 + NBW:]
    pcnts = [(c + (REG_PAD - 1)) & (-REG_PAD) for c in rawcnt]

    for k in range(NBW):
        row_v[pl.ds(0, 16)] = jnp.where(lanes == 0, starts[k],
                                        jnp.where(lanes == 1, pcnts[k], 0))
        pltpu.sync_copy(row_v, meta_hbm.at[mybins[k]])

    stages = ((s0, d0, e0), (s1, d1, e1), (s2, d2, e2))
    bins = mybins

    def chunk(ci, carry):
        base = pl.multiple_of(ci * CH, 8)
        pltpu.sync_copy(src_hbm.at[pl.ds(base, CH)], sbuf.at[pl.ds(0, CH)])
        pltpu.sync_copy(dst_hbm.at[pl.ds(base, CH)], dbuf.at[pl.ds(0, CH)])

        def grp(g, c2):
            cur = list(c2[0:NBW])
            fl = list(c2[NBW:])
            off = g * 16
            sv = sbuf[pl.ds(off, 16)]
            dv = dbuf[pl.ds(off, 16)]
            ev = (base + off) + lanes
            bv = _bin_of(dv)
            for k in range(NBW):
                ss, ds_, es = stages[k]
                m = bv == bins[k]
                mi = m.astype(jnp.int32)
                cs = plsc.cumsum(mi)
                pos = cur[k] + cs - mi
                plsc.store_scatter(ss, [pos], sv, mask=m)
                plsc.store_scatter(ds_, [pos], dv, mask=m)
                plsc.store_scatter(es, [pos], ev, mask=m)
                cur[k] = cur[k] + cs[15]
                do = cur[k] >= FLUSH

                @pl.when(do)
                def _flush(k=k, ss=ss, ds_=ds_, es=es):
                    tgt = pl.multiple_of(starts[k] + fl[k], 8)
                    pltpu.sync_copy(ss.at[pl.ds(0, FLUSH)],
                                    psrc_hbm.at[pl.ds(tgt, FLUSH)])
                    pltpu.sync_copy(ds_.at[pl.ds(0, FLUSH)],
                                    pdst_hbm.at[pl.ds(tgt, FLUSH)])
                    pltpu.sync_copy(es.at[pl.ds(0, FLUSH)],
                                    peid_hbm.at[pl.ds(tgt, FLUSH)])
                    ss[pl.ds(0, 16)] = ss[pl.ds(FLUSH, 16)]
                    ds_[pl.ds(0, 16)] = ds_[pl.ds(FLUSH, 16)]
                    es[pl.ds(0, 16)] = es[pl.ds(FLUSH, 16)]

                doi = do.astype(jnp.int32)
                cur[k] = cur[k] - FLUSH * doi
                fl[k] = fl[k] + FLUSH * doi
            return tuple(cur) + tuple(fl)

        return lax.fori_loop(0, CH // 16, grp, carry)

    fin = lax.fori_loop(0, nch, chunk, (0,) * (2 * NBW))

    curs = fin[0:NBW]
    fls = fin[NBW:]
    for k in range(NBW):
        ss, ds_, es = stages[k]
        cur = curs[k]
        trash_dst = BS * (bins[k] + 1)
        zvec = jnp.zeros((16,), jnp.int32)
        tvec = jnp.full((16,), trash_dst, jnp.int32)
        for j in range(REG_PAD // 16):
            pos = (cur + 16 * j) + lanes
            plsc.store_scatter(ss, [pos], zvec)
            plsc.store_scatter(ds_, [pos], tvec)
            plsc.store_scatter(es, [pos], zvec)
        rem = (cur + (REG_PAD - 1)) & (-REG_PAD)
        for sz in (1024, 512, 256, 128):
            o = pl.multiple_of(rem & (-(2 * sz)), 8)

            @pl.when((rem & sz) != 0)
            def _tail(k=k, ss=ss, ds_=ds_, es=es, sz=sz, o=o):
                tgt = pl.multiple_of(starts[k] + fls[k] + o, 8)
                pltpu.sync_copy(ss.at[pl.ds(o, sz)], psrc_hbm.at[pl.ds(tgt, sz)])
                pltpu.sync_copy(ds_.at[pl.ds(o, sz)], pdst_hbm.at[pl.ds(tgt, sz)])
                pltpu.sync_copy(es.at[pl.ds(o, sz)], peid_hbm.at[pl.ds(tgt, sz)])


@functools.partial(
    pl.kernel,
    mesh=_mesh,
    compiler_params=_sc_params,
    out_type=(
        jax.ShapeDtypeStruct((NPAD, H), jnp.float32),
        jax.ShapeDtypeStruct((NPAD, H), jnp.float32),
        jax.ShapeDtypeStruct((NPAD, H), jnp.float32),
        jax.ShapeDtypeStruct((NPAD, H), jnp.float32),
        jax.ShapeDtypeStruct((NPAD, 16), jnp.float32),
    ),
    scratch_types=[
        pltpu.VMEM((BS + 1, H), jnp.float32),
        pltpu.VMEM((BS + 1, H), jnp.float32),
        pltpu.VMEM((BS + 1, H), jnp.float32),
        pltpu.VMEM((BS + 1, H), jnp.float32),
        pltpu.VMEM((BS + 1, 16), jnp.float32),
        pltpu.VMEM((BS, H), jnp.float32),
        pltpu.VMEM((2, KCH, H), jnp.float32),
        pltpu.VMEM((2, KCH, H), jnp.float32),
        pltpu.VMEM((2, KCH + 16,), jnp.int32),
        pltpu.VMEM((2, KCH + 16,), jnp.int32),
        pltpu.VMEM((2, KCH + 16,), jnp.int32),
        pltpu.VMEM((16,), jnp.int32),
        pltpu.SemaphoreType.DMA,
        pltpu.SemaphoreType.DMA,
        pltpu.SemaphoreType.DMA,
    ],
)
def _sc_agg(a_hbm, b_hbm, c_hbm, psrc_hbm, pdst_hbm, peid_hbm, meta_hbm,
            s_hbm, q_hbm, mx_hbm, mn_hbm, deg_hbm,
            accS, accQ, accM, accN, accD, bsl, arows, crows,
            sidx, didx, eidx, row_v, semi, sema, semc):
    w = _wid()
    zero16 = jnp.zeros((16,), jnp.float32)
    ones16 = jnp.ones((16,), jnp.float32)
    neg16 = jnp.full((16,), NEG, jnp.float32)
    pos16 = jnp.full((16,), POS, jnp.float32)

    for which in range(NBW):
        b = NBW * w + which
        lo = b * BS
        pltpu.sync_copy(meta_hbm.at[b], row_v)
        mrow = row_v[pl.ds(0, 16)]
        bstart = mrow[0]
        pcnt = mrow[1]
        ntrips = lax.shift_right_logical(pcnt, 5)

        def zrow(r, c):
            for t in range(H // 16):
                accS[r, pl.ds(t * 16, 16)] = zero16
                accQ[r, pl.ds(t * 16, 16)] = zero16
                accM[r, pl.ds(t * 16, 16)] = neg16
                accN[r, pl.ds(t * 16, 16)] = pos16
            accD[r, pl.ds(0, 16)] = zero16
            return c

        lax.fori_loop(0, BS + 1, zrow, 0)

        pltpu.sync_copy(b_hbm.at[pl.ds(lo, BS)], bsl)

        def issue(t, slot):
            off = pl.multiple_of(bstart + t * KCH, 8)
            pltpu.sync_copy(psrc_hbm.at[pl.ds(off, KCH)],
                            sidx.at[slot, pl.ds(0, KCH)])
            pltpu.sync_copy(pdst_hbm.at[pl.ds(off, KCH)],
                            didx.at[slot, pl.ds(0, KCH)])
            pltpu.sync_copy(peid_hbm.at[pl.ds(off, KCH)],
                            eidx.at[slot, pl.ds(0, KCH)])
            pltpu.async_copy(a_hbm.at[sidx.at[slot, pl.ds(0, KCH)]],
                             arows.at[slot], sema)
            pltpu.async_copy(c_hbm.at[eidx.at[slot, pl.ds(0, KCH)]],
                             crows.at[slot], semc)

        @pl.when(ntrips > 0)
        def _prime():
            issue(0, 0)

        def trip(t, c):
            slot = t & 1
            pltpu.make_async_copy(a_hbm.at[sidx.at[slot, pl.ds(0, KCH)]],
                                  arows.at[slot], sema).wait()
            pltpu.make_async_copy(c_hbm.at[eidx.at[slot, pl.ds(0, KCH)]],
                                  crows.at[slot], semc).wait()

            @pl.when(t + 1 < ntrips)
            def _next():
                issue(t + 1, 1 - slot)

            def edge(e, c2):
                dloc = _lane(didx.at[slot], e) - lo
                dlob = jnp.minimum(dloc, BS - 1)
                for t8 in range(H // 16):
                    cs = pl.ds(t8 * 16, 16)
                    av = arows[slot, e, cs]
                    cv = crows[slot, e, cs]
                    bv = bsl[dlob, cs]
                    m = jnp.maximum(av + bv + cv, 0.0)
                    plsc.addupdate(accS.at[dloc, cs], m)
                    plsc.addupdate(accQ.at[dloc, cs], m * m)
                    accM[dloc, cs] = jnp.maximum(accM[dloc, cs], m)
                    accN[dloc, cs] = jnp.minimum(accN[dloc, cs], m)
                plsc.addupdate(accD.at[dloc, pl.ds(0, 16)], ones16)
                return c2

            lax.fori_loop(0, KCH, edge, 0)
            return c

        lax.fori_loop(0, ntrips, trip, 0)

        pltpu.sync_copy(accS.at[pl.ds(0, BS)], s_hbm.at[pl.ds(lo, BS)])
        pltpu.sync_copy(accQ.at[pl.ds(0, BS)], q_hbm.at[pl.ds(lo, BS)])
        pltpu.sync_copy(accM.at[pl.ds(0, BS)], mx_hbm.at[pl.ds(lo, BS)])
        pltpu.sync_copy(accN.at[pl.ds(0, BS)], mn_hbm.at[pl.ds(lo, BS)])
        pltpu.sync_copy(accD.at[pl.ds(0, BS)], deg_hbm.at[pl.ds(lo, BS)])


def _tc_in(x, W, b):
    def body(x_ref, w_ref, b_ref, o_ref):
        o_ref[...] = jnp.dot(x_ref[...], w_ref[...],
                             preferred_element_type=jnp.float32) + b_ref[...]

    return pl.pallas_call(
        body,
        grid=(NPAD // 512,),
        in_specs=[
            pl.BlockSpec((512, H), lambda i: (i, 0)),
            pl.BlockSpec((H, H), lambda i: (0, 0)),
            pl.BlockSpec((1, H), lambda i: (0, 0)),
        ],
        out_specs=pl.BlockSpec((512, H), lambda i: (i, 0)),
        out_shape=jax.ShapeDtypeStruct((NPAD, H), jnp.float32),
    )(x, W, b)


def _tc_ab(h, Wa, Wb):
    def body(h_ref, wa_ref, wb_ref, a_ref, b_ref):
        hb = h_ref[...]
        a_ref[...] = jnp.dot(hb, wa_ref[...], preferred_element_type=jnp.float32)
        b_ref[...] = jnp.dot(hb, wb_ref[...], preferred_element_type=jnp.float32)

    return pl.pallas_call(
        body,
        grid=(NPAD // 512,),
        in_specs=[
            pl.BlockSpec((512, H), lambda i: (i, 0)),
            pl.BlockSpec((H, H), lambda i: (0, 0)),
            pl.BlockSpec((H, H), lambda i: (0, 0)),
        ],
        out_specs=[
            pl.BlockSpec((512, H), lambda i: (i, 0)),
            pl.BlockSpec((512, H), lambda i: (i, 0)),
        ],
        out_shape=[
            jax.ShapeDtypeStruct((NPAD, H), jnp.float32),
            jax.ShapeDtypeStruct((NPAD, H), jnp.float32),
        ],
    )(h, Wa, Wb)


def _tc_c(ea, Wc, bp):
    def body(e_ref, w_ref, b_ref, o_ref):
        o_ref[...] = jnp.dot(e_ref[...], w_ref[...],
                             preferred_element_type=jnp.float32) + b_ref[...]

    return pl.pallas_call(
        body,
        grid=(E // 2000,),
        in_specs=[
            pl.BlockSpec((2000, DE), lambda i: (i, 0)),
            pl.BlockSpec((DE, H), lambda i: (0, 0)),
            pl.BlockSpec((1, H), lambda i: (0, 0)),
        ],
        out_specs=pl.BlockSpec((2000, H), lambda i: (i, 0)),
        out_shape=jax.ShapeDtypeStruct((E, H), jnp.float32),
    )(ea, Wc, bp)


def _tc_delta(deg16):
    def body(d_ref, o_ref):
        d = d_ref[...]
        rows = lax.broadcasted_iota(jnp.int32, d.shape, 0)
        cols = lax.broadcasted_iota(jnp.int32, d.shape, 1)
        valid = (rows < N) & (cols == 0)
        logd = jnp.where(valid, jnp.log(d + 1.0), 0.0)
        o_ref[0, 0] = jnp.sum(logd) / N

    return pl.pallas_call(
        body,
        grid=(1,),
        in_specs=[pl.BlockSpec((NPAD, 16), lambda i: (0, 0))],
        out_specs=pl.BlockSpec((1, 1), lambda i: (0, 0), memory_space=pltpu.SMEM),
        out_shape=jax.ShapeDtypeStruct((1, 1), jnp.float32),
    )(deg16)


def _tc_post(h, S, Q, Mx, Mn, deg16, delta, Wpost, bpost):
    def body(h_ref, s_ref, q_ref, mx_ref, mn_ref, d_ref, del_ref, w_ref,
             b_ref, o_ref):
        hb = h_ref[...]
        deg = d_ref[...][:, 0:1]
        dlt = del_ref[0, 0]
        logd = jnp.log(deg + 1.0)
        amp = logd / dlt
        att = dlt / jnp.maximum(logd, 1e-5)
        cnt = jnp.maximum(deg, 1.0)
        mask = deg > 0.0
        mean = s_ref[...] / cnt
        msq = q_ref[...] / cnt
        std = jnp.sqrt(jnp.maximum(msq - mean * mean, 0.0) + 1e-5)
        mx = jnp.where(mask, mx_ref[...], 0.0)
        mn = jnp.where(mask, mn_ref[...], 0.0)
        parts = [hb, mean, mx, mn, std,
                 mean * amp, mx * amp, mn * amp, std * amp,
                 mean * att, mx * att, mn * att, std * att]
        acc = b_ref[...]
        for j, p in enumerate(parts):
            acc = acc + jnp.dot(p, w_ref[pl.ds(j * H, H), :],
                                preferred_element_type=jnp.float32)
        o_ref[...] = hb + jnp.maximum(acc, 0.0)

    blk = 512
    return pl.pallas_call(
        body,
        grid=(NPAD // blk,),
        in_specs=[
            pl.BlockSpec((blk, H), lambda i: (i, 0)),
            pl.BlockSpec((blk, H), lambda i: (i, 0)),
            pl.BlockSpec((blk, H), lambda i: (i, 0)),
            pl.BlockSpec((blk, H), lambda i: (i, 0)),
            pl.BlockSpec((blk, H), lambda i: (i, 0)),
            pl.BlockSpec((blk, 16), lambda i: (i, 0)),
            pl.BlockSpec((1, 1), lambda i: (0, 0), memory_space=pltpu.SMEM),
            pl.BlockSpec((13 * H, H), lambda i: (0, 0)),
            pl.BlockSpec((1, H), lambda i: (0, 0)),
        ],
        out_specs=pl.BlockSpec((blk, H), lambda i: (i, 0)),
        out_shape=jax.ShapeDtypeStruct((NPAD, H), jnp.float32),
    )(h, S, Q, Mx, Mn, deg16, delta, Wpost, bpost)


def _tc_head(h, Wp1, bp1, Wp2, bp2):
    def body(h_ref, w1_ref, b1_ref, w2_ref, b2_ref, o_ref):
        t = jnp.maximum(jnp.dot(h_ref[...], w1_ref[...],
                                preferred_element_type=jnp.float32)
                        + b1_ref[...], 0.0)
        o_ref[...] = jnp.maximum(jnp.dot(t, w2_ref[...],
                                         preferred_element_type=jnp.float32)
                                 + b2_ref[...], 0.0)

    return pl.pallas_call(
        body,
        grid=(NPAD // 512,),
        in_specs=[
            pl.BlockSpec((512, H), lambda i: (i, 0)),
            pl.BlockSpec((H, H), lambda i: (0, 0)),
            pl.BlockSpec((1, H), lambda i: (0, 0)),
            pl.BlockSpec((H, TGT), lambda i: (0, 0)),
            pl.BlockSpec((1, TGT), lambda i: (0, 0)),
        ],
        out_specs=pl.BlockSpec((512, TGT), lambda i: (i, 0)),
        out_shape=jax.ShapeDtypeStruct((NPAD, TGT), jnp.float32),
    )(h, Wp1, bp1, Wp2, bp2)


def kernel(x, edge_attr, W_in, b_in, Wpre, bpre, Wpost, bpost, Wp1, bp1,
           Wp2, bp2, edge_index):
    src = edge_index[0]
    dst = edge_index[1]

    counts = _sc_count(dst)
    psrc, pdst, peid, meta = _sc_place(src, dst, counts)

    xpad = jnp.pad(x, ((0, NPAD - N), (0, 0)))
    h = _tc_in(xpad, W_in, b_in.reshape(1, H))

    delta = None
    for l in range(L):
        Wa = Wpre[l, :H]
        Wb = Wpre[l, H:2 * H]
        Wc = Wpre[l, 2 * H:]
        A, B = _tc_ab(h, Wa, Wb)
        C = _tc_c(edge_attr, Wc, bpre[l].reshape(1, H))
        S, Q, Mx, Mn, deg16 = _sc_agg(A, B, C, psrc, pdst, peid, meta)
        if delta is None:
            delta = _tc_delta(deg16)
        h = _tc_post(h, S, Q, Mx, Mn, deg16, delta, Wpost[l],
                     bpost[l].reshape(1, H))

    y = _tc_head(h, Wp1, bp1.reshape(1, H), Wp2, bp2.reshape(1, TGT))
    return y[:N]


# speedup vs baseline: 2.3246x; 1.2202x over previous
import functools

import jax
import jax.numpy as jnp
from jax import lax
from jax.experimental import pallas as pl
from jax.experimental.pallas import tpu as pltpu
from jax.experimental.pallas import tpu_sc as plsc

N = 10000
E = 320000
H = 128
DE = 16
L = 3
TGT = 128

NPAD = 12288
NBINS = 96
BS = 128
NBW = 3
NW = 32
KCH = 32
REG_PAD = 128
EP = E + NBINS * REG_PAD
STAGE = 1152
FLUSH = 1024
NEG = -3.0e38
POS = 3.0e38

_mesh = plsc.VectorSubcoreMesh(core_axis_name="c", subcore_axis_name="s")
_sc_params = pltpu.CompilerParams(needs_layout_passes=False)


def _wid():
    return lax.axis_index("s") * 2 + lax.axis_index("c")


def _bin_of(v16):
    return lax.shift_right_logical(v16, 7)


def _lane(ref, i):
    return ref[pl.ds(i, 16)][0]


@functools.partial(
    pl.kernel,
    mesh=_mesh,
    compiler_params=_sc_params,
    out_type=jax.ShapeDtypeStruct((NBINS, 16), jnp.int32),
    scratch_types=[
        pltpu.VMEM((2016,), jnp.int32),
        pltpu.VMEM((16,), jnp.int32),
    ],
)
def _sc_count(dst_hbm, counts_hbm, dbuf, row_v):
    w = _wid()
    mybins = [NBW * w + k for k in range(NBW)]
    CH = 2000
    nch = E // CH

    def chunk(ci, carry):
        base = pl.multiple_of(ci * CH, 8)
        pltpu.sync_copy(dst_hbm.at[pl.ds(base, CH)], dbuf.at[pl.ds(0, CH)])

        def grp(g, c2):
            v = dbuf[pl.ds(g * 16, 16)]
            b = _bin_of(v)
            return tuple(c2[k] + jnp.sum((b == mybins[k]).astype(jnp.int32))
                         for k in range(NBW))

        return lax.fori_loop(0, CH // 16, grp, carry)

    cnts = lax.fori_loop(0, nch, chunk, (0,) * NBW)
    lanes = lax.iota(jnp.int32, 16)
    for k in range(NBW):
        row_v[pl.ds(0, 16)] = jnp.where(lanes == 0, cnts[k], 0)
        pltpu.sync_copy(row_v, counts_hbm.at[mybins[k]])


@functools.partial(
    pl.kernel,
    mesh=_mesh,
    compiler_params=_sc_params,
    out_type=(
        jax.ShapeDtypeStruct((EP,), jnp.int32),
        jax.ShapeDtypeStruct((EP,), jnp.int32),
        jax.ShapeDtypeStruct((EP,), jnp.int32),
        jax.ShapeDtypeStruct((NBINS, 16), jnp.int32),
    ),
    scratch_types=[
        pltpu.VMEM((NBINS, 16), jnp.int32),
        pltpu.VMEM((2016,), jnp.int32),
        pltpu.VMEM((2016,), jnp.int32),
        pltpu.VMEM((STAGE,), jnp.int32),
        pltpu.VMEM((STAGE,), jnp.int32),
        pltpu.VMEM((STAGE,), jnp.int32),
        pltpu.VMEM((STAGE,), jnp.int32),
        pltpu.VMEM((STAGE,), jnp.int32),
        pltpu.VMEM((STAGE,), jnp.int32),
        pltpu.VMEM((STAGE,), jnp.int32),
        pltpu.VMEM((STAGE,), jnp.int32),
        pltpu.VMEM((STAGE,), jnp.int32),
        pltpu.VMEM((16,), jnp.int32),
    ],
)
def _sc_place(src_hbm, dst_hbm, counts_hbm, psrc_hbm, pdst_hbm, peid_hbm,
              meta_hbm, cnts_v, sbuf, dbuf, s0, d0, e0, s1, d1, e1,
              s2, d2, e2, row_v):
    w = _wid()
    mybins = [NBW * w + k for k in range(NBW)]
    CH = 2000
    nch = E // CH
    lanes = lax.iota(jnp.int32, 16)

    pltpu.sync_copy(counts_hbm, cnts_v)

    def scan_bins(b, carry):
        s = carry[0]
        sts = list(carry[1:1 + NBW])
        cns = list(carry[1 + NBW:])
        c = cnts_v[b, pl.ds(0, 16)][0]
        cp = (c + (REG_PAD - 1)) & (-REG_PAD)
        for k in range(NBW):
            sts[k] = jnp.where(b == mybins[k], s, sts[k])
            cns[k] = jnp.where(b == mybins[k], c, cns[k])
        return (s + cp,) + tuple(sts) + tuple(cns)

    res = lax.fori_loop(0, NBINS, scan_bins, (0,) * (1 + 2 * NBW))
    starts = res[1:1 + NBW]
    rawcnt = res[1 + NBW:]
    pcnts = [(c + (REG_PAD - 1)) & (-REG_PAD) for c in rawcnt]

    for k in range(NBW):
        row_v[pl.ds(0, 16)] = jnp.where(lanes == 0, starts[k],
                                        jnp.where(lanes == 1, pcnts[k], 0))
        pltpu.sync_copy(row_v, meta_hbm.at[mybins[k]])

    stages = ((s0, d0, e0), (s1, d1, e1), (s2, d2, e2))
    bins = mybins

    def chunk(ci, carry):
        base = pl.multiple_of(ci * CH, 8)
        pltpu.sync_copy(src_hbm.at[pl.ds(base, CH)], sbuf.at[pl.ds(0, CH)])
        pltpu.sync_copy(dst_hbm.at[pl.ds(base, CH)], dbuf.at[pl.ds(0, CH)])

        def grp(g, c2):
            cur = list(c2[0:NBW])
            fl = list(c2[NBW:])
            off = g * 16
            sv = sbuf[pl.ds(off, 16)]
            dv = dbuf[pl.ds(off, 16)]
            ev = (base + off) + lanes
            bv = _bin_of(dv)
            for k in range(NBW):
                ss, ds_, es = stages[k]
                m = bv == bins[k]
                mi = m.astype(jnp.int32)
                cs = plsc.cumsum(mi)
                pos = cur[k] + cs - mi
                plsc.store_scatter(ss, [pos], sv, mask=m)
                plsc.store_scatter(ds_, [pos], dv, mask=m)
                plsc.store_scatter(es, [pos], ev, mask=m)
                cur[k] = cur[k] + cs[15]
                do = cur[k] >= FLUSH

                @pl.when(do)
                def _flush(k=k, ss=ss, ds_=ds_, es=es):
                    tgt = pl.multiple_of(starts[k] + fl[k], 8)
                    pltpu.sync_copy(ss.at[pl.ds(0, FLUSH)],
                                    psrc_hbm.at[pl.ds(tgt, FLUSH)])
                    pltpu.sync_copy(ds_.at[pl.ds(0, FLUSH)],
                                    pdst_hbm.at[pl.ds(tgt, FLUSH)])
                    pltpu.sync_copy(es.at[pl.ds(0, FLUSH)],
                                    peid_hbm.at[pl.ds(tgt, FLUSH)])
                    ss[pl.ds(0, 16)] = ss[pl.ds(FLUSH, 16)]
                    ds_[pl.ds(0, 16)] = ds_[pl.ds(FLUSH, 16)]
                    es[pl.ds(0, 16)] = es[pl.ds(FLUSH, 16)]

                doi = do.astype(jnp.int32)
                cur[k] = cur[k] - FLUSH * doi
                fl[k] = fl[k] + FLUSH * doi
            return tuple(cur) + tuple(fl)

        return lax.fori_loop(0, CH // 16, grp, carry)

    fin = lax.fori_loop(0, nch, chunk, (0,) * (2 * NBW))

    curs = fin[0:NBW]
    fls = fin[NBW:]
    for k in range(NBW):
        ss, ds_, es = stages[k]
        cur = curs[k]
        trash_dst = BS * (bins[k] + 1)
        zvec = jnp.zeros((16,), jnp.int32)
        tvec = jnp.full((16,), trash_dst, jnp.int32)
        for j in range(REG_PAD // 16):
            pos = (cur + 16 * j) + lanes
            plsc.store_scatter(ss, [pos], zvec)
            plsc.store_scatter(ds_, [pos], tvec)
            plsc.store_scatter(es, [pos], zvec)
        rem = (cur + (REG_PAD - 1)) & (-REG_PAD)
        for sz in (1024, 512, 256, 128):
            o = pl.multiple_of(rem & (-(2 * sz)), 8)

            @pl.when((rem & sz) != 0)
            def _tail(k=k, ss=ss, ds_=ds_, es=es, sz=sz, o=o):
                tgt = pl.multiple_of(starts[k] + fls[k] + o, 8)
                pltpu.sync_copy(ss.at[pl.ds(o, sz)], psrc_hbm.at[pl.ds(tgt, sz)])
                pltpu.sync_copy(ds_.at[pl.ds(o, sz)], pdst_hbm.at[pl.ds(tgt, sz)])
                pltpu.sync_copy(es.at[pl.ds(o, sz)], peid_hbm.at[pl.ds(tgt, sz)])


@functools.partial(
    pl.kernel,
    mesh=_mesh,
    compiler_params=_sc_params,
    out_type=(
        jax.ShapeDtypeStruct((NPAD, H), jnp.float32),
        jax.ShapeDtypeStruct((NPAD, H), jnp.float32),
        jax.ShapeDtypeStruct((NPAD, H), jnp.float32),
        jax.ShapeDtypeStruct((NPAD, H), jnp.float32),
        jax.ShapeDtypeStruct((NPAD, 16), jnp.float32),
    ),
    scratch_types=[
        pltpu.VMEM((BS + 1, H), jnp.float32),
        pltpu.VMEM((BS + 1, H), jnp.float32),
        pltpu.VMEM((BS + 1, H), jnp.float32),
        pltpu.VMEM((BS + 1, H), jnp.float32),
        pltpu.VMEM((BS + 1, 16), jnp.float32),
        pltpu.VMEM((BS, H), jnp.float32),
        pltpu.VMEM((KCH, H), jnp.float32),
        pltpu.VMEM((KCH, H), jnp.float32),
        pltpu.VMEM((KCH, H), jnp.float32),
        pltpu.VMEM((KCH, H), jnp.float32),
        pltpu.VMEM((KCH,), jnp.int32),
        pltpu.VMEM((KCH,), jnp.int32),
        pltpu.VMEM((KCH + 16,), jnp.int32),
        pltpu.VMEM((KCH + 16,), jnp.int32),
        pltpu.VMEM((KCH,), jnp.int32),
        pltpu.VMEM((KCH,), jnp.int32),
        pltpu.VMEM((KCH + 16,), jnp.int32),
        pltpu.VMEM((16,), jnp.int32),
        pltpu.SemaphoreType.DMA,
        pltpu.SemaphoreType.DMA,
        pltpu.SemaphoreType.DMA,
        pltpu.SemaphoreType.DMA,
        pltpu.SemaphoreType.DMA,
        pltpu.SemaphoreType.DMA,
    ],
)
def _sc_agg(a_hbm, b_hbm, c_hbm, psrc_hbm, pdst_hbm, peid_hbm, meta_hbm,
            s_hbm, q_hbm, mx_hbm, mn_hbm, deg_hbm,
            accS, accQ, accM, accN, accD, bsl, ar0, ar1, cr0, cr1,
            si0, si1, di0, di1, ei0, ei1, dcur, row_v,
            smi0, smi1, sma0, sma1, smc0, smc1):
    w = _wid()
    zero16 = jnp.zeros((16,), jnp.float32)
    ones16 = jnp.ones((16,), jnp.float32)
    neg16 = jnp.full((16,), NEG, jnp.float32)
    pos16 = jnp.full((16,), POS, jnp.float32)

    ar = (ar0, ar1)
    cr = (cr0, cr1)
    si = (si0, si1)
    di = (di0, di1)
    ei = (ei0, ei1)
    smi = (smi0, smi1)
    sma = (sma0, sma1)
    smc = (smc0, smc1)

    for which in range(NBW):
        b = NBW * w + which
        lo = b * BS
        pltpu.sync_copy(meta_hbm.at[b], row_v)
        mrow = row_v[pl.ds(0, 16)]
        bstart = mrow[0]
        pcnt = mrow[1]
        ntrips = lax.shift_right_logical(pcnt, 5)

        def zrow(r, c):
            for t in range(H // 16):
                accS[r, pl.ds(t * 16, 16)] = zero16
                accQ[r, pl.ds(t * 16, 16)] = zero16
                accM[r, pl.ds(t * 16, 16)] = neg16
                accN[r, pl.ds(t * 16, 16)] = pos16
            accD[r, pl.ds(0, 16)] = zero16
            return c

        lax.fori_loop(0, BS + 1, zrow, 0)

        pltpu.sync_copy(b_hbm.at[pl.ds(lo, BS)], bsl)

        def idx_descs(t, s):
            off = pl.multiple_of(bstart + t * KCH, 8)
            return (
                (psrc_hbm.at[pl.ds(off, KCH)], si[s], smi[s]),
                (pdst_hbm.at[pl.ds(off, KCH)], di[s].at[pl.ds(0, KCH)], smi[s]),
                (peid_hbm.at[pl.ds(off, KCH)], ei[s], smi[s]),
            )

        def issue_idx(t, s):
            for sref, dref, sem in idx_descs(t, s):
                pltpu.async_copy(sref, dref, sem)

        def wait_idx(t, s):
            for sref, dref, sem in idx_descs(t, s):
                pltpu.make_async_copy(sref, dref, sem).wait()

        def issue_g(s):
            pltpu.async_copy(a_hbm.at[si[s]], ar[s], sma[s])
            pltpu.async_copy(c_hbm.at[ei[s]], cr[s], smc[s])

        def wait_g(s):
            pltpu.make_async_copy(a_hbm.at[si[s]], ar[s], sma[s]).wait()
            pltpu.make_async_copy(c_hbm.at[ei[s]], cr[s], smc[s]).wait()

        @pl.when(ntrips > 0)
        def _prime():
            issue_idx(0, 0)
            wait_idx(0, 0)
            issue_g(0)
            issue_idx(1, 1)

        def compute(s):
            ars = ar[s]
            crs = cr[s]

            def edge(i, c2):
                for j in range(2):
                    e = 2 * i + j
                    dloc = _lane(dcur, e) - lo
                    dlob = jnp.minimum(dloc, BS - 1)
                    for t8 in range(H // 16):
                        cs = pl.ds(t8 * 16, 16)
                        av = ars[e, cs]
                        cv = crs[e, cs]
                        bv = bsl[dlob, cs]
                        m = jnp.maximum(av + bv + cv, 0.0)
                        plsc.addupdate(accS.at[dloc, cs], m)
                        plsc.addupdate(accQ.at[dloc, cs], m * m)
                        accM[dloc, cs] = jnp.maximum(accM[dloc, cs], m)
                        accN[dloc, cs] = jnp.minimum(accN[dloc, cs], m)
                    plsc.addupdate(accD.at[dloc, pl.ds(0, 16)], ones16)
                return c2

            lax.fori_loop(0, KCH // 2, edge, 0)

        def super_trip(u, c):
            for s in range(2):
                t = 2 * u + s
                wait_g(s)

                @pl.when(t + 1 < ntrips)
                def _w(s=s, t=t):
                    wait_idx(t + 1, 1 - s)

                for j in range(KCH // 16):
                    dcur[pl.ds(j * 16, 16)] = di[s][pl.ds(j * 16, 16)]

                @pl.when(t + 1 < ntrips)
                def _g(s=s, t=t):
                    issue_g(1 - s)

                @pl.when(t + 2 < ntrips)
                def _i(s=s, t=t):
                    issue_idx(t + 2, s)

                compute(s)
            return c

        lax.fori_loop(0, lax.shift_right_logical(ntrips, 1), super_trip, 0)

        pltpu.sync_copy(accS.at[pl.ds(0, BS)], s_hbm.at[pl.ds(lo, BS)])
        pltpu.sync_copy(accQ.at[pl.ds(0, BS)], q_hbm.at[pl.ds(lo, BS)])
        pltpu.sync_copy(accM.at[pl.ds(0, BS)], mx_hbm.at[pl.ds(lo, BS)])
        pltpu.sync_copy(accN.at[pl.ds(0, BS)], mn_hbm.at[pl.ds(lo, BS)])
        pltpu.sync_copy(accD.at[pl.ds(0, BS)], deg_hbm.at[pl.ds(lo, BS)])


def _tc_in(x, W, b):
    def body(x_ref, w_ref, b_ref, o_ref):
        o_ref[...] = jnp.dot(x_ref[...], w_ref[...],
                             preferred_element_type=jnp.float32) + b_ref[...]

    return pl.pallas_call(
        body,
        grid=(NPAD // 512,),
        in_specs=[
            pl.BlockSpec((512, H), lambda i: (i, 0)),
            pl.BlockSpec((H, H), lambda i: (0, 0)),
            pl.BlockSpec((1, H), lambda i: (0, 0)),
        ],
        out_specs=pl.BlockSpec((512, H), lambda i: (i, 0)),
        out_shape=jax.ShapeDtypeStruct((NPAD, H), jnp.float32),
    )(x, W, b)


def _tc_ab(h, Wa, Wb):
    def body(h_ref, wa_ref, wb_ref, a_ref, b_ref):
        hb = h_ref[...]
        a_ref[...] = jnp.dot(hb, wa_ref[...], preferred_element_type=jnp.float32)
        b_ref[...] = jnp.dot(hb, wb_ref[...], preferred_element_type=jnp.float32)

    return pl.pallas_call(
        body,
        grid=(NPAD // 512,),
        in_specs=[
            pl.BlockSpec((512, H), lambda i: (i, 0)),
            pl.BlockSpec((H, H), lambda i: (0, 0)),
            pl.BlockSpec((H, H), lambda i: (0, 0)),
        ],
        out_specs=[
            pl.BlockSpec((512, H), lambda i: (i, 0)),
            pl.BlockSpec((512, H), lambda i: (i, 0)),
        ],
        out_shape=[
            jax.ShapeDtypeStruct((NPAD, H), jnp.float32),
            jax.ShapeDtypeStruct((NPAD, H), jnp.float32),
        ],
    )(h, Wa, Wb)


def _tc_c(ea, Wc, bp):
    def body(e_ref, w_ref, b_ref, o_ref):
        o_ref[...] = jnp.dot(e_ref[...], w_ref[...],
                             preferred_element_type=jnp.float32) + b_ref[...]

    return pl.pallas_call(
        body,
        grid=(E // 2000,),
        in_specs=[
            pl.BlockSpec((2000, DE), lambda i: (i, 0)),
            pl.BlockSpec((DE, H), lambda i: (0, 0)),
            pl.BlockSpec((1, H), lambda i: (0, 0)),
        ],
        out_specs=pl.BlockSpec((2000, H), lambda i: (i, 0)),
        out_shape=jax.ShapeDtypeStruct((E, H), jnp.float32),
    )(ea, Wc, bp)


def _tc_delta(deg16):
    def body(d_ref, o_ref):
        d = d_ref[...]
        rows = lax.broadcasted_iota(jnp.int32, d.shape, 0)
        cols = lax.broadcasted_iota(jnp.int32, d.shape, 1)
        valid = (rows < N) & (cols == 0)
        logd = jnp.where(valid, jnp.log(d + 1.0), 0.0)
        o_ref[0, 0] = jnp.sum(logd) / N

    return pl.pallas_call(
        body,
        grid=(1,),
        in_specs=[pl.BlockSpec((NPAD, 16), lambda i: (0, 0))],
        out_specs=pl.BlockSpec((1, 1), lambda i: (0, 0), memory_space=pltpu.SMEM),
        out_shape=jax.ShapeDtypeStruct((1, 1), jnp.float32),
    )(deg16)


def _tc_post(h, S, Q, Mx, Mn, deg16, delta, Wpost, bpost):
    def body(h_ref, s_ref, q_ref, mx_ref, mn_ref, d_ref, del_ref, w_ref,
             b_ref, o_ref):
        hb = h_ref[...]
        deg = d_ref[...][:, 0:1]
        dlt = del_ref[0, 0]
        logd = jnp.log(deg + 1.0)
        amp = logd / dlt
        att = dlt / jnp.maximum(logd, 1e-5)
        cnt = jnp.maximum(deg, 1.0)
        mask = deg > 0.0
        mean = s_ref[...] / cnt
        msq = q_ref[...] / cnt
        std = jnp.sqrt(jnp.maximum(msq - mean * mean, 0.0) + 1e-5)
        mx = jnp.where(mask, mx_ref[...], 0.0)
        mn = jnp.where(mask, mn_ref[...], 0.0)
        parts = [hb, mean, mx, mn, std,
                 mean * amp, mx * amp, mn * amp, std * amp,
                 mean * att, mx * att, mn * att, std * att]
        acc = b_ref[...]
        for j, p in enumerate(parts):
            acc = acc + jnp.dot(p, w_ref[pl.ds(j * H, H), :],
                                preferred_element_type=jnp.float32)
        o_ref[...] = hb + jnp.maximum(acc, 0.0)

    blk = 512
    return pl.pallas_call(
        body,
        grid=(NPAD // blk,),
        in_specs=[
            pl.BlockSpec((blk, H), lambda i: (i, 0)),
            pl.BlockSpec((blk, H), lambda i: (i, 0)),
            pl.BlockSpec((blk, H), lambda i: (i, 0)),
            pl.BlockSpec((blk, H), lambda i: (i, 0)),
            pl.BlockSpec((blk, H), lambda i: (i, 0)),
            pl.BlockSpec((blk, 16), lambda i: (i, 0)),
            pl.BlockSpec((1, 1), lambda i: (0, 0), memory_space=pltpu.SMEM),
            pl.BlockSpec((13 * H, H), lambda i: (0, 0)),
            pl.BlockSpec((1, H), lambda i: (0, 0)),
        ],
        out_specs=pl.BlockSpec((blk, H), lambda i: (i, 0)),
        out_shape=jax.ShapeDtypeStruct((NPAD, H), jnp.float32),
    )(h, S, Q, Mx, Mn, deg16, delta, Wpost, bpost)


def _tc_head(h, Wp1, bp1, Wp2, bp2):
    def body(h_ref, w1_ref, b1_ref, w2_ref, b2_ref, o_ref):
        t = jnp.maximum(jnp.dot(h_ref[...], w1_ref[...],
                                preferred_element_type=jnp.float32)
                        + b1_ref[...], 0.0)
        o_ref[...] = jnp.maximum(jnp.dot(t, w2_ref[...],
                                         preferred_element_type=jnp.float32)
                                 + b2_ref[...], 0.0)

    return pl.pallas_call(
        body,
        grid=(NPAD // 512,),
        in_specs=[
            pl.BlockSpec((512, H), lambda i: (i, 0)),
            pl.BlockSpec((H, H), lambda i: (0, 0)),
            pl.BlockSpec((1, H), lambda i: (0, 0)),
            pl.BlockSpec((H, TGT), lambda i: (0, 0)),
            pl.BlockSpec((1, TGT), lambda i: (0, 0)),
        ],
        out_specs=pl.BlockSpec((512, TGT), lambda i: (i, 0)),
        out_shape=jax.ShapeDtypeStruct((NPAD, TGT), jnp.float32),
    )(h, Wp1, bp1, Wp2, bp2)


def kernel(x, edge_attr, W_in, b_in, Wpre, bpre, Wpost, bpost, Wp1, bp1,
           Wp2, bp2, edge_index):
    src = edge_index[0]
    dst = edge_index[1]

    counts = _sc_count(dst)
    psrc, pdst, peid, meta = _sc_place(src, dst, counts)

    xpad = jnp.pad(x, ((0, NPAD - N), (0, 0)))
    h = _tc_in(xpad, W_in, b_in.reshape(1, H))

    delta = None
    for l in range(L):
        Wa = Wpre[l, :H]
        Wb = Wpre[l, H:2 * H]
        Wc = Wpre[l, 2 * H:]
        A, B = _tc_ab(h, Wa, Wb)
        C = _tc_c(edge_attr, Wc, bpre[l].reshape(1, H))
        S, Q, Mx, Mn, deg16 = _sc_agg(A, B, C, psrc, pdst, peid, meta)
        if delta is None:
            delta = _tc_delta(deg16)
        h = _tc_post(h, S, Q, Mx, Mn, deg16, delta, Wpost[l],
                     bpost[l].reshape(1, H))

    y = _tc_head(h, Wp1, bp1.reshape(1, H), Wp2, bp2.reshape(1, TGT))
    return y[:N]


# speedup vs baseline: 3.6515x; 1.5708x over previous
import functools

import jax
import jax.numpy as jnp
from jax import lax
from jax.experimental import pallas as pl
from jax.experimental.pallas import tpu as pltpu
from jax.experimental.pallas import tpu_sc as plsc

N = 10000
E = 320000
H = 128
DE = 16
L = 3
TGT = 128

NPAD = 12288
NBINS = 96
BS = 128
NBW = 3
NW = 32
KCH = 32
REG_PAD = 128
EP = E + NBINS * REG_PAD
STAGE = 1152
FLUSH = 1024
NEG = -3.0e38
POS = 3.0e38

_mesh = plsc.VectorSubcoreMesh(core_axis_name="c", subcore_axis_name="s")
_sc_params = pltpu.CompilerParams(needs_layout_passes=False)


def _wid():
    return lax.axis_index("s") * 2 + lax.axis_index("c")


def _bin_of(v16):
    return lax.shift_right_logical(v16, 7)


def _lane(ref, i):
    return ref[pl.ds(i, 16)][0]


@functools.partial(
    pl.kernel,
    mesh=_mesh,
    compiler_params=_sc_params,
    out_type=jax.ShapeDtypeStruct((NBINS, 16), jnp.int32),
    scratch_types=[
        pltpu.VMEM((2016,), jnp.int32),
        pltpu.VMEM((16,), jnp.int32),
    ],
)
def _sc_count(dst_hbm, counts_hbm, dbuf, row_v):
    w = _wid()
    mybins = [NBW * w + k for k in range(NBW)]
    CH = 2000
    nch = E // CH

    def chunk(ci, carry):
        base = pl.multiple_of(ci * CH, 8)
        pltpu.sync_copy(dst_hbm.at[pl.ds(base, CH)], dbuf.at[pl.ds(0, CH)])

        def grp(g, c2):
            v = dbuf[pl.ds(g * 16, 16)]
            b = _bin_of(v)
            return tuple(c2[k] + jnp.sum((b == mybins[k]).astype(jnp.int32))
                         for k in range(NBW))

        return lax.fori_loop(0, CH // 16, grp, carry)

    cnts = lax.fori_loop(0, nch, chunk, (0,) * NBW)
    lanes = lax.iota(jnp.int32, 16)
    for k in range(NBW):
        row_v[pl.ds(0, 16)] = jnp.where(lanes == 0, cnts[k], 0)
        pltpu.sync_copy(row_v, counts_hbm.at[mybins[k]])


@functools.partial(
    pl.kernel,
    mesh=_mesh,
    compiler_params=_sc_params,
    out_type=(
        jax.ShapeDtypeStruct((EP,), jnp.int32),
        jax.ShapeDtypeStruct((EP,), jnp.int32),
        jax.ShapeDtypeStruct((EP,), jnp.int32),
        jax.ShapeDtypeStruct((NBINS, 16), jnp.int32),
    ),
    scratch_types=[
        pltpu.VMEM((NBINS, 16), jnp.int32),
        pltpu.VMEM((2016,), jnp.int32),
        pltpu.VMEM((2016,), jnp.int32),
        pltpu.VMEM((STAGE,), jnp.int32),
        pltpu.VMEM((STAGE,), jnp.int32),
        pltpu.VMEM((STAGE,), jnp.int32),
        pltpu.VMEM((STAGE,), jnp.int32),
        pltpu.VMEM((STAGE,), jnp.int32),
        pltpu.VMEM((STAGE,), jnp.int32),
        pltpu.VMEM((STAGE,), jnp.int32),
        pltpu.VMEM((STAGE,), jnp.int32),
        pltpu.VMEM((STAGE,), jnp.int32),
        pltpu.VMEM((16,), jnp.int32),
    ],
)
def _sc_place(src_hbm, dst_hbm, counts_hbm, psrc_hbm, pdst_hbm, peid_hbm,
              meta_hbm, cnts_v, sbuf, dbuf, s0, d0, e0, s1, d1, e1,
              s2, d2, e2, row_v):
    w = _wid()
    mybins = [NBW * w + k for k in range(NBW)]
    CH = 2000
    nch = E // CH
    lanes = lax.iota(jnp.int32, 16)

    pltpu.sync_copy(counts_hbm, cnts_v)

    def scan_bins(b, carry):
        s = carry[0]
        sts = list(carry[1:1 + NBW])
        cns = list(carry[1 + NBW:])
        c = cnts_v[b, pl.ds(0, 16)][0]
        cp = (c + (REG_PAD - 1)) & (-REG_PAD)
        for k in range(NBW):
            sts[k] = jnp.where(b == mybins[k], s, sts[k])
            cns[k] = jnp.where(b == mybins[k], c, cns[k])
        return (s + cp,) + tuple(sts) + tuple(cns)

    res = lax.fori_loop(0, NBINS, scan_bins, (0,) * (1 + 2 * NBW))
    starts = res[1:1 + NBW]
    rawcnt = res[1 + NBW:]
    pcnts = [(c + (REG_PAD - 1)) & (-REG_PAD) for c in rawcnt]

    for k in range(NBW):
        row_v[pl.ds(0, 16)] = jnp.where(lanes == 0, starts[k],
                                        jnp.where(lanes == 1, pcnts[k], 0))
        pltpu.sync_copy(row_v, meta_hbm.at[mybins[k]])

    stages = ((s0, d0, e0), (s1, d1, e1), (s2, d2, e2))
    bins = mybins

    def chunk(ci, carry):
        base = pl.multiple_of(ci * CH, 8)
        pltpu.sync_copy(src_hbm.at[pl.ds(base, CH)], sbuf.at[pl.ds(0, CH)])
        pltpu.sync_copy(dst_hbm.at[pl.ds(base, CH)], dbuf.at[pl.ds(0, CH)])

        def grp(g, c2):
            cur = list(c2[0:NBW])
            fl = list(c2[NBW:])
            off = g * 16
            sv = sbuf[pl.ds(off, 16)]
            dv = dbuf[pl.ds(off, 16)]
            ev = (base + off) + lanes
            bv = _bin_of(dv)
            for k in range(NBW):
                ss, ds_, es = stages[k]
                m = bv == bins[k]
                mi = m.astype(jnp.int32)
                cs = plsc.cumsum(mi)
                pos = cur[k] + cs - mi
                plsc.store_scatter(ss, [pos], sv, mask=m)
                plsc.store_scatter(ds_, [pos], dv, mask=m)
                plsc.store_scatter(es, [pos], ev, mask=m)
                cur[k] = cur[k] + cs[15]
                do = cur[k] >= FLUSH

                @pl.when(do)
                def _flush(k=k, ss=ss, ds_=ds_, es=es):
                    tgt = pl.multiple_of(starts[k] + fl[k], 8)
                    pltpu.sync_copy(ss.at[pl.ds(0, FLUSH)],
                                    psrc_hbm.at[pl.ds(tgt, FLUSH)])
                    pltpu.sync_copy(ds_.at[pl.ds(0, FLUSH)],
                                    pdst_hbm.at[pl.ds(tgt, FLUSH)])
                    pltpu.sync_copy(es.at[pl.ds(0, FLUSH)],
                                    peid_hbm.at[pl.ds(tgt, FLUSH)])
                    ss[pl.ds(0, 16)] = ss[pl.ds(FLUSH, 16)]
                    ds_[pl.ds(0, 16)] = ds_[pl.ds(FLUSH, 16)]
                    es[pl.ds(0, 16)] = es[pl.ds(FLUSH, 16)]

                doi = do.astype(jnp.int32)
                cur[k] = cur[k] - FLUSH * doi
                fl[k] = fl[k] + FLUSH * doi
            return tuple(cur) + tuple(fl)

        return lax.fori_loop(0, CH // 16, grp, carry)

    fin = lax.fori_loop(0, nch, chunk, (0,) * (2 * NBW))

    curs = fin[0:NBW]
    fls = fin[NBW:]
    for k in range(NBW):
        ss, ds_, es = stages[k]
        cur = curs[k]
        trash_dst = BS * (bins[k] + 1)
        zvec = jnp.zeros((16,), jnp.int32)
        tvec = jnp.full((16,), trash_dst, jnp.int32)
        for j in range(REG_PAD // 16):
            pos = (cur + 16 * j) + lanes
            plsc.store_scatter(ss, [pos], zvec)
            plsc.store_scatter(ds_, [pos], tvec)
            plsc.store_scatter(es, [pos], zvec)
        rem = (cur + (REG_PAD - 1)) & (-REG_PAD)
        for sz in (1024, 512, 256, 128):
            o = pl.multiple_of(rem & (-(2 * sz)), 8)

            @pl.when((rem & sz) != 0)
            def _tail(k=k, ss=ss, ds_=ds_, es=es, sz=sz, o=o):
                tgt = pl.multiple_of(starts[k] + fls[k] + o, 8)
                pltpu.sync_copy(ss.at[pl.ds(o, sz)], psrc_hbm.at[pl.ds(tgt, sz)])
                pltpu.sync_copy(ds_.at[pl.ds(o, sz)], pdst_hbm.at[pl.ds(tgt, sz)])
                pltpu.sync_copy(es.at[pl.ds(o, sz)], peid_hbm.at[pl.ds(tgt, sz)])


@functools.partial(
    pl.kernel,
    mesh=_mesh,
    compiler_params=_sc_params,
    out_type=(
        jax.ShapeDtypeStruct((NPAD, H), jnp.float32),
        jax.ShapeDtypeStruct((NPAD, H), jnp.float32),
        jax.ShapeDtypeStruct((NPAD, H), jnp.float32),
        jax.ShapeDtypeStruct((NPAD, H), jnp.float32),
        jax.ShapeDtypeStruct((NPAD, 16), jnp.float32),
    ),
    scratch_types=[
        pltpu.VMEM((BS + 1, H), jnp.float32),
        pltpu.VMEM((BS + 1, H), jnp.float32),
        pltpu.VMEM((BS + 1, H), jnp.float32),
        pltpu.VMEM((BS + 1, H), jnp.float32),
        pltpu.VMEM((BS + 1, 16), jnp.float32),
        pltpu.VMEM((BS, H), jnp.float32),
        pltpu.VMEM((KCH, H), jnp.float32),
        pltpu.VMEM((KCH, H), jnp.float32),
        pltpu.VMEM((KCH, H), jnp.float32),
        pltpu.VMEM((KCH, H), jnp.float32),
        pltpu.VMEM((KCH,), jnp.int32),
        pltpu.VMEM((KCH,), jnp.int32),
        pltpu.VMEM((KCH + 16,), jnp.int32),
        pltpu.VMEM((KCH + 16,), jnp.int32),
        pltpu.VMEM((KCH,), jnp.int32),
        pltpu.VMEM((KCH,), jnp.int32),
        pltpu.VMEM((KCH + 16,), jnp.int32),
        pltpu.VMEM((16,), jnp.int32),
        pltpu.SemaphoreType.DMA,
        pltpu.SemaphoreType.DMA,
        pltpu.SemaphoreType.DMA,
        pltpu.SemaphoreType.DMA,
        pltpu.SemaphoreType.DMA,
        pltpu.SemaphoreType.DMA,
    ],
)
def _sc_agg(a_hbm, b_hbm, c_hbm, psrc_hbm, pdst_hbm, peid_hbm, meta_hbm,
            s_hbm, q_hbm, mx_hbm, mn_hbm, deg_hbm,
            accS, accQ, accM, accN, accD, bsl, ar0, ar1, cr0, cr1,
            si0, si1, di0, di1, ei0, ei1, dcur, row_v,
            smi0, smi1, sma0, sma1, smc0, smc1):
    w = _wid()
    zero16 = jnp.zeros((16,), jnp.float32)
    ones16 = jnp.ones((16,), jnp.float32)
    neg16 = jnp.full((16,), NEG, jnp.float32)
    pos16 = jnp.full((16,), POS, jnp.float32)

    ar = (ar0, ar1)
    cr = (cr0, cr1)
    si = (si0, si1)
    di = (di0, di1)
    ei = (ei0, ei1)
    smi = (smi0, smi1)
    sma = (sma0, sma1)
    smc = (smc0, smc1)

    for which in range(NBW):
        b = NBW * w + which
        lo = b * BS
        pltpu.sync_copy(meta_hbm.at[b], row_v)
        mrow = row_v[pl.ds(0, 16)]
        bstart = mrow[0]
        pcnt = mrow[1]
        ntrips = lax.shift_right_logical(pcnt, 5)

        def zrow(r, c):
            for t in range(H // 16):
                accS[r, pl.ds(t * 16, 16)] = zero16
                accQ[r, pl.ds(t * 16, 16)] = zero16
                accM[r, pl.ds(t * 16, 16)] = neg16
                accN[r, pl.ds(t * 16, 16)] = pos16
            accD[r, pl.ds(0, 16)] = zero16
            return c

        lax.fori_loop(0, BS + 1, zrow, 0)

        pltpu.sync_copy(b_hbm.at[pl.ds(lo, BS)], bsl)

        def idx_descs(t, s):
            off = pl.multiple_of(bstart + t * KCH, 8)
            return (
                (psrc_hbm.at[pl.ds(off, KCH)], si[s], smi[s]),
                (pdst_hbm.at[pl.ds(off, KCH)], di[s].at[pl.ds(0, KCH)], smi[s]),
                (peid_hbm.at[pl.ds(off, KCH)], ei[s], smi[s]),
            )

        def issue_idx(t, s):
            for sref, dref, sem in idx_descs(t, s):
                pltpu.async_copy(sref, dref, sem)

        def wait_idx(t, s):
            for sref, dref, sem in idx_descs(t, s):
                pltpu.make_async_copy(sref, dref, sem).wait()

        def issue_g(s):
            pltpu.async_copy(a_hbm.at[si[s]], ar[s], sma[s])
            pltpu.async_copy(c_hbm.at[ei[s]], cr[s], smc[s])

        def wait_g(s):
            pltpu.make_async_copy(a_hbm.at[si[s]], ar[s], sma[s]).wait()
            pltpu.make_async_copy(c_hbm.at[ei[s]], cr[s], smc[s]).wait()

        @pl.when(ntrips > 0)
        def _prime():
            issue_idx(0, 0)
            wait_idx(0, 0)
            issue_g(0)
            issue_idx(1, 1)

        def compute(s):
            ars = ar[s]
            crs = cr[s]
            UE = 4

            def edge(i, c2):
                g = i * UE
                dv = dcur[pl.ds(g, 16)] - lo
                dvb = jnp.minimum(dv, BS - 1)
                for j in range(UE):
                    e = g + j
                    dloc = dv[j]
                    dlob = dvb[j]
                    nb = H // 16
                    css = [pl.ds(t8 * 16, 16) for t8 in range(nb)]
                    av = [ars[e, cs] for cs in css]
                    cv = [crs[e, cs] for cs in css]
                    bv = [bsl[dlob, cs] for cs in css]
                    m = [jnp.maximum(av[t] + bv[t] + cv[t], 0.0)
                         for t in range(nb)]
                    for t in range(nb):
                        plsc.addupdate(accS.at[dloc, css[t]], m[t])
                    for t in range(nb):
                        plsc.addupdate(accQ.at[dloc, css[t]], m[t] * m[t])
                    oldM = [accM[dloc, cs] for cs in css]
                    for t in range(nb):
                        accM[dloc, css[t]] = jnp.maximum(oldM[t], m[t])
                    oldN = [accN[dloc, cs] for cs in css]
                    for t in range(nb):
                        accN[dloc, css[t]] = jnp.minimum(oldN[t], m[t])
                    plsc.addupdate(accD.at[dloc, pl.ds(0, 16)], ones16)
                return c2

            lax.fori_loop(0, KCH // UE, edge, 0)

        def super_trip(u, c):
            for s in range(2):
                t = 2 * u + s
                wait_g(s)

                @pl.when(t + 1 < ntrips)
                def _w(s=s, t=t):
                    wait_idx(t + 1, 1 - s)

                for j in range(KCH // 16):
                    dcur[pl.ds(j * 16, 16)] = di[s][pl.ds(j * 16, 16)]

                @pl.when(t + 1 < ntrips)
                def _g(s=s, t=t):
                    issue_g(1 - s)

                @pl.when(t + 2 < ntrips)
                def _i(s=s, t=t):
                    issue_idx(t + 2, s)

                compute(s)
            return c

        lax.fori_loop(0, lax.shift_right_logical(ntrips, 1), super_trip, 0)

        pltpu.sync_copy(accS.at[pl.ds(0, BS)], s_hbm.at[pl.ds(lo, BS)])
        pltpu.sync_copy(accQ.at[pl.ds(0, BS)], q_hbm.at[pl.ds(lo, BS)])
        pltpu.sync_copy(accM.at[pl.ds(0, BS)], mx_hbm.at[pl.ds(lo, BS)])
        pltpu.sync_copy(accN.at[pl.ds(0, BS)], mn_hbm.at[pl.ds(lo, BS)])
        pltpu.sync_copy(accD.at[pl.ds(0, BS)], deg_hbm.at[pl.ds(lo, BS)])


def _tc_in(x, W, b):
    def body(x_ref, w_ref, b_ref, o_ref):
        o_ref[...] = jnp.dot(x_ref[...], w_ref[...],
                             preferred_element_type=jnp.float32) + b_ref[...]

    return pl.pallas_call(
        body,
        grid=(NPAD // 512,),
        in_specs=[
            pl.BlockSpec((512, H), lambda i: (i, 0)),
            pl.BlockSpec((H, H), lambda i: (0, 0)),
            pl.BlockSpec((1, H), lambda i: (0, 0)),
        ],
        out_specs=pl.BlockSpec((512, H), lambda i: (i, 0)),
        out_shape=jax.ShapeDtypeStruct((NPAD, H), jnp.float32),
    )(x, W, b)


def _tc_ab(h, Wa, Wb):
    def body(h_ref, wa_ref, wb_ref, a_ref, b_ref):
        hb = h_ref[...]
        a_ref[...] = jnp.dot(hb, wa_ref[...], preferred_element_type=jnp.float32)
        b_ref[...] = jnp.dot(hb, wb_ref[...], preferred_element_type=jnp.float32)

    return pl.pallas_call(
        body,
        grid=(NPAD // 512,),
        in_specs=[
            pl.BlockSpec((512, H), lambda i: (i, 0)),
            pl.BlockSpec((H, H), lambda i: (0, 0)),
            pl.BlockSpec((H, H), lambda i: (0, 0)),
        ],
        out_specs=[
            pl.BlockSpec((512, H), lambda i: (i, 0)),
            pl.BlockSpec((512, H), lambda i: (i, 0)),
        ],
        out_shape=[
            jax.ShapeDtypeStruct((NPAD, H), jnp.float32),
            jax.ShapeDtypeStruct((NPAD, H), jnp.float32),
        ],
    )(h, Wa, Wb)


def _tc_c(ea, Wc, bp):
    def body(e_ref, w_ref, b_ref, o_ref):
        o_ref[...] = jnp.dot(e_ref[...], w_ref[...],
                             preferred_element_type=jnp.float32) + b_ref[...]

    return pl.pallas_call(
        body,
        grid=(E // 2000,),
        in_specs=[
            pl.BlockSpec((2000, DE), lambda i: (i, 0)),
            pl.BlockSpec((DE, H), lambda i: (0, 0)),
            pl.BlockSpec((1, H), lambda i: (0, 0)),
        ],
        out_specs=pl.BlockSpec((2000, H), lambda i: (i, 0)),
        out_shape=jax.ShapeDtypeStruct((E, H), jnp.float32),
    )(ea, Wc, bp)


def _tc_delta(deg16):
    def body(d_ref, o_ref):
        d = d_ref[...]
        rows = lax.broadcasted_iota(jnp.int32, d.shape, 0)
        cols = lax.broadcasted_iota(jnp.int32, d.shape, 1)
        valid = (rows < N) & (cols == 0)
        logd = jnp.where(valid, jnp.log(d + 1.0), 0.0)
        o_ref[0, 0] = jnp.sum(logd) / N

    return pl.pallas_call(
        body,
        grid=(1,),
        in_specs=[pl.BlockSpec((NPAD, 16), lambda i: (0, 0))],
        out_specs=pl.BlockSpec((1, 1), lambda i: (0, 0), memory_space=pltpu.SMEM),
        out_shape=jax.ShapeDtypeStruct((1, 1), jnp.float32),
    )(deg16)


def _tc_post(h, S, Q, Mx, Mn, deg16, delta, Wpost, bpost):
    def body(h_ref, s_ref, q_ref, mx_ref, mn_ref, d_ref, del_ref, w_ref,
             b_ref, o_ref):
        hb = h_ref[...]
        deg = d_ref[...][:, 0:1]
        dlt = del_ref[0, 0]
        logd = jnp.log(deg + 1.0)
        amp = logd / dlt
        att = dlt / jnp.maximum(logd, 1e-5)
        cnt = jnp.maximum(deg, 1.0)
        mask = deg > 0.0
        mean = s_ref[...] / cnt
        msq = q_ref[...] / cnt
        std = jnp.sqrt(jnp.maximum(msq - mean * mean, 0.0) + 1e-5)
        mx = jnp.where(mask, mx_ref[...], 0.0)
        mn = jnp.where(mask, mn_ref[...], 0.0)
        parts = [hb, mean, mx, mn, std,
                 mean * amp, mx * amp, mn * amp, std * amp,
                 mean * att, mx * att, mn * att, std * att]
        acc = b_ref[...]
        for j, p in enumerate(parts):
            acc = acc + jnp.dot(p, w_ref[pl.ds(j * H, H), :],
                                preferred_element_type=jnp.float32)
        o_ref[...] = hb + jnp.maximum(acc, 0.0)

    blk = 512
    return pl.pallas_call(
        body,
        grid=(NPAD // blk,),
        in_specs=[
            pl.BlockSpec((blk, H), lambda i: (i, 0)),
            pl.BlockSpec((blk, H), lambda i: (i, 0)),
            pl.BlockSpec((blk, H), lambda i: (i, 0)),
            pl.BlockSpec((blk, H), lambda i: (i, 0)),
            pl.BlockSpec((blk, H), lambda i: (i, 0)),
            pl.BlockSpec((blk, 16), lambda i: (i, 0)),
            pl.BlockSpec((1, 1), lambda i: (0, 0), memory_space=pltpu.SMEM),
            pl.BlockSpec((13 * H, H), lambda i: (0, 0)),
            pl.BlockSpec((1, H), lambda i: (0, 0)),
        ],
        out_specs=pl.BlockSpec((blk, H), lambda i: (i, 0)),
        out_shape=jax.ShapeDtypeStruct((NPAD, H), jnp.float32),
    )(h, S, Q, Mx, Mn, deg16, delta, Wpost, bpost)


def _tc_head(h, Wp1, bp1, Wp2, bp2):
    def body(h_ref, w1_ref, b1_ref, w2_ref, b2_ref, o_ref):
        t = jnp.maximum(jnp.dot(h_ref[...], w1_ref[...],
                                preferred_element_type=jnp.float32)
                        + b1_ref[...], 0.0)
        o_ref[...] = jnp.maximum(jnp.dot(t, w2_ref[...],
                                         preferred_element_type=jnp.float32)
                                 + b2_ref[...], 0.0)

    return pl.pallas_call(
        body,
        grid=(NPAD // 512,),
        in_specs=[
            pl.BlockSpec((512, H), lambda i: (i, 0)),
            pl.BlockSpec((H, H), lambda i: (0, 0)),
            pl.BlockSpec((1, H), lambda i: (0, 0)),
            pl.BlockSpec((H, TGT), lambda i: (0, 0)),
            pl.BlockSpec((1, TGT), lambda i: (0, 0)),
        ],
        out_specs=pl.BlockSpec((512, TGT), lambda i: (i, 0)),
        out_shape=jax.ShapeDtypeStruct((NPAD, TGT), jnp.float32),
    )(h, Wp1, bp1, Wp2, bp2)


def kernel(x, edge_attr, W_in, b_in, Wpre, bpre, Wpost, bpost, Wp1, bp1,
           Wp2, bp2, edge_index):
    src = edge_index[0]
    dst = edge_index[1]

    counts = _sc_count(dst)
    psrc, pdst, peid, meta = _sc_place(src, dst, counts)

    xpad = jnp.pad(x, ((0, NPAD - N), (0, 0)))
    h = _tc_in(xpad, W_in, b_in.reshape(1, H))

    delta = None
    for l in range(L):
        Wa = Wpre[l, :H]
        Wb = Wpre[l, H:2 * H]
        Wc = Wpre[l, 2 * H:]
        A, B = _tc_ab(h, Wa, Wb)
        C = _tc_c(edge_attr, Wc, bpre[l].reshape(1, H))
        S, Q, Mx, Mn, deg16 = _sc_agg(A, B, C, psrc, pdst, peid, meta)
        if delta is None:
            delta = _tc_delta(deg16)
        h = _tc_post(h, S, Q, Mx, Mn, deg16, delta, Wpost[l],
                     bpost[l].reshape(1, H))

    y = _tc_head(h, Wp1, bp1.reshape(1, H), Wp2, bp2.reshape(1, TGT))
    return y[:N]
